# Initial kernel scaffold; baseline (speedup 1.0000x reference)
import functools

import jax
import jax.numpy as jnp
from jax import lax
from jax.experimental import pallas as pl
from jax.experimental.pallas import tpu as pltpu
from jax.experimental.pallas import tpu_sc as plsc

NC = 2
NS = 16
NW = NC * NS
K = 128


def _sc_mesh():
    return plsc.VectorSubcoreMesh(core_axis_name="c", subcore_axis_name="s")


def _deg_kernel(npad, ch):
    sl = npad // NS

    @functools.partial(
        pl.kernel,
        mesh=_sc_mesh(),
        out_type=jax.ShapeDtypeStruct((NC, npad), jnp.float32),
        scratch_types=[
            pltpu.VMEM_SHARED((npad,), jnp.float32),
            pltpu.VMEM((ch, K), jnp.int32),
            pltpu.VMEM((K,), jnp.float32),
            pltpu.SemaphoreType.DMA,
        ],
    )
    def deg_kernel(dst_hbm, zeros_hbm, out_hbm, acc_sh, idx_v, ones_v, sem):
        c = lax.axis_index("c")
        s = lax.axis_index("s")
        wid = s * NC + c
        pltpu.sync_copy(zeros_hbm.at[pl.ds(s * sl, sl)], acc_sh.at[pl.ds(s * sl, sl)])

        @pl.loop(0, K, step=16)
        def _(i):
            ones_v[pl.ds(i, 16)] = jnp.full((16,), 1.0, jnp.float32)

        pltpu.sync_copy(dst_hbm.at[wid], idx_v)
        plsc.subcore_barrier()

        @pl.loop(0, ch)
        def _(j):
            pltpu.sync_copy(ones_v, acc_sh.at[idx_v.at[j]], add=True)

        plsc.subcore_barrier()
        pltpu.sync_copy(acc_sh.at[pl.ds(s * sl, sl)], out_hbm.at[c, pl.ds(s * sl, sl)])

    return deg_kernel


def _segsum_kernel(npad, ch, h):
    sl = npad // NS

    @functools.partial(
        pl.kernel,
        mesh=_sc_mesh(),
        out_type=jax.ShapeDtypeStruct((NC, npad, h), jnp.float32),
        scratch_types=[
            pltpu.VMEM_SHARED((npad, h), jnp.float32),
            pltpu.VMEM((ch, K), jnp.int32),
            pltpu.VMEM((ch, K), jnp.int32),
            pltpu.VMEM((K, h), jnp.float32),
            pltpu.SemaphoreType.DMA,
        ],
    )
    def seg_kernel(y_hbm, src_hbm, dst_hbm, zeros_hbm, out_hbm,
                   acc_sh, src_v, dst_v, rows_v, gsem):
        c = lax.axis_index("c")
        s = lax.axis_index("s")
        wid = s * NC + c
        pltpu.sync_copy(zeros_hbm.at[pl.ds(s * sl, sl)], acc_sh.at[pl.ds(s * sl, sl)])
        pltpu.sync_copy(src_hbm.at[wid], src_v)
        pltpu.sync_copy(dst_hbm.at[wid], dst_v)
        plsc.subcore_barrier()

        @pl.loop(0, ch)
        def _(j):
            pltpu.async_copy(y_hbm.at[src_v.at[j]], rows_v, gsem).wait()
            pltpu.sync_copy(rows_v, acc_sh.at[dst_v.at[j]], add=True)

        plsc.subcore_barrier()
        pltpu.sync_copy(acc_sh.at[pl.ds(s * sl, sl)], out_hbm.at[c, pl.ds(s * sl, sl)])

    return seg_kernel


def _tc1_body(x_ref, wg_ref, wps_ref, bps_ref, xw_ref, zsem_ref):
    xb = x_ref[...]
    xw_ref[...] = jnp.dot(xb, wg_ref[...], preferred_element_type=jnp.float32)
    zsem_ref[...] = (
        jnp.dot(xb, wps_ref[...], preferred_element_type=jnp.float32) + bps_ref[...]
    )


def _tc2_body(degt_ref, xw_ref, y_ref):
    deg = degt_ref[:, 0:1] + degt_ref[:, 1:2] + 1.0
    dis = lax.rsqrt(deg)
    y_ref[...] = xw_ref[...] * dis


def _tc3_body(degt_ref, ap_ref, xw_ref, zsem_ref, wpt_ref, bpt_ref,
              wcls_ref, bcls_ref, bgcn_ref, zt_ref, lg_ref, an_ref):
    acc = ap_ref[0] + ap_ref[1]
    deg = degt_ref[:, 0:1] + degt_ref[:, 1:2] + 1.0
    dis = lax.rsqrt(deg)
    h = acc * dis + xw_ref[...] * (dis * dis) + bgcn_ref[...]
    h = jnp.maximum(h, 0.0)
    zt = jnp.dot(h, wpt_ref[...], preferred_element_type=jnp.float32) + bpt_ref[...]
    lg = jnp.dot(zt, wcls_ref[...], preferred_element_type=jnp.float32) + bcls_ref[...]
    diff = zt - zsem_ref[...]
    an_ref[...] = jnp.sqrt(jnp.sum(diff * diff, axis=1, keepdims=True))
    zt_ref[...] = zt
    lg_ref[...] = lg


def kernel(x, edge_index, W_gcn, b_gcn, W_pt, b_pt, W_ps, b_ps, W_cls, b_cls):
    n, d = x.shape
    h = W_gcn.shape[1]
    a = W_pt.shape[1]
    ncls = W_cls.shape[1]
    e = edge_index.shape[1]

    npad = ((n + 1 + 127) // 128) * 128
    ch = -(-e // (NW * K))
    ep = NW * ch * K

    src = edge_index[0]
    dst = edge_index[1]
    padi = jnp.full((ep - e,), n, jnp.int32)
    srcp = jnp.concatenate([src, padi]).reshape(NW, ch, K)
    dstp = jnp.concatenate([dst, padi]).reshape(NW, ch, K)
    xp = jnp.concatenate([x, jnp.zeros((npad - n, d), x.dtype)], axis=0)
    zeros_n = jnp.zeros((npad,), jnp.float32)
    zeros_nh = jnp.zeros((npad, h), jnp.float32)

    bn = npad // 8

    xw, z_sem = pl.pallas_call(
        _tc1_body,
        grid=(8,),
        in_specs=[
            pl.BlockSpec((bn, d), lambda i: (i, 0)),
            pl.BlockSpec((d, h), lambda i: (0, 0)),
            pl.BlockSpec((d, a), lambda i: (0, 0)),
            pl.BlockSpec((1, a), lambda i: (0, 0)),
        ],
        out_specs=[
            pl.BlockSpec((bn, h), lambda i: (i, 0)),
            pl.BlockSpec((bn, a), lambda i: (i, 0)),
        ],
        out_shape=[
            jax.ShapeDtypeStruct((npad, h), jnp.float32),
            jax.ShapeDtypeStruct((npad, a), jnp.float32),
        ],
    )(xp, W_gcn, W_ps, b_ps.reshape(1, a))

    deg_parts = _deg_kernel(npad, ch)(dstp, zeros_n)
    deg_t = deg_parts.T

    y = pl.pallas_call(
        _tc2_body,
        grid=(8,),
        in_specs=[
            pl.BlockSpec((bn, NC), lambda i: (i, 0)),
            pl.BlockSpec((bn, h), lambda i: (i, 0)),
        ],
        out_specs=pl.BlockSpec((bn, h), lambda i: (i, 0)),
        out_shape=jax.ShapeDtypeStruct((npad, h), jnp.float32),
    )(deg_t, xw)

    acc_parts = _segsum_kernel(npad, ch, h)(y, srcp, dstp, zeros_nh)

    z_topo, logits, anomaly = pl.pallas_call(
        _tc3_body,
        grid=(8,),
        in_specs=[
            pl.BlockSpec((bn, NC), lambda i: (i, 0)),
            pl.BlockSpec((NC, bn, h), lambda i: (0, i, 0)),
            pl.BlockSpec((bn, h), lambda i: (i, 0)),
            pl.BlockSpec((bn, a), lambda i: (i, 0)),
            pl.BlockSpec((h, a), lambda i: (0, 0)),
            pl.BlockSpec((1, a), lambda i: (0, 0)),
            pl.BlockSpec((a, ncls), lambda i: (0, 0)),
            pl.BlockSpec((1, ncls), lambda i: (0, 0)),
            pl.BlockSpec((1, h), lambda i: (0, 0)),
        ],
        out_specs=[
            pl.BlockSpec((bn, a), lambda i: (i, 0)),
            pl.BlockSpec((bn, ncls), lambda i: (i, 0)),
            pl.BlockSpec((bn, 1), lambda i: (i, 0)),
        ],
        out_shape=[
            jax.ShapeDtypeStruct((npad, a), jnp.float32),
            jax.ShapeDtypeStruct((npad, ncls), jnp.float32),
            jax.ShapeDtypeStruct((npad, 1), jnp.float32),
        ],
    )(deg_t, acc_parts, xw, z_sem, W_pt, b_pt.reshape(1, a),
      W_cls, b_cls.reshape(1, ncls), b_gcn.reshape(1, h))

    return (logits[:n], anomaly[:n, 0], z_topo[:n], z_sem[:n])


# speedup vs baseline: 26.1621x; 26.1621x over previous
import functools

import jax
import jax.numpy as jnp
from jax import lax
from jax.experimental import pallas as pl
from jax.experimental.pallas import tpu as pltpu
from jax.experimental.pallas import tpu_sc as plsc

NC = 2
NS = 16
NW = NC * NS
K = 128


def _sc_mesh():
    return plsc.VectorSubcoreMesh(core_axis_name="c", subcore_axis_name="s")


_SC_PARAMS = pltpu.CompilerParams(use_tc_tiling_on_sc=False)


def _deg_kernel(npad, ch):
    sl = npad // NS

    @functools.partial(
        pl.kernel,
        mesh=_sc_mesh(),
        out_type=jax.ShapeDtypeStruct((NC * npad,), jnp.float32),
        compiler_params=_SC_PARAMS,
        scratch_types=[
            pltpu.VMEM_SHARED((npad,), jnp.float32),
            pltpu.VMEM((ch, K), jnp.int32),
            pltpu.VMEM((K,), jnp.float32),
            pltpu.VMEM((sl,), jnp.float32),
            pltpu.SemaphoreType.DMA,
        ],
    )
    def deg_kernel(dst_hbm, zeros_hbm, out_hbm, acc_sh, idx_v, ones_v, zst_v, sem):
        c = lax.axis_index("c")
        s = lax.axis_index("s")
        wid = s * NC + c
        pltpu.sync_copy(zeros_hbm.at[pl.ds(s * sl, sl)], zst_v)
        pltpu.sync_copy(zst_v, acc_sh.at[pl.ds(s * sl, sl)])

        @pl.loop(0, K, step=16)
        def _(i):
            ones_v[pl.ds(i, 16)] = jnp.full((16,), 1.0, jnp.float32)

        pltpu.sync_copy(dst_hbm.at[wid], idx_v)
        plsc.subcore_barrier()

        @pl.loop(0, ch)
        def _(j):
            pltpu.sync_copy(ones_v, acc_sh.at[idx_v.at[j]], add=True)

        plsc.subcore_barrier()
        pltpu.sync_copy(acc_sh.at[pl.ds(s * sl, sl)], zst_v)
        pltpu.sync_copy(zst_v, out_hbm.at[pl.ds(c * npad + s * sl, sl)])

    return deg_kernel


def _segsum_kernel(npad, ch, h):
    sl = npad // NS

    @functools.partial(
        pl.kernel,
        mesh=_sc_mesh(),
        out_type=jax.ShapeDtypeStruct((NC, npad, h), jnp.float32),
        compiler_params=_SC_PARAMS,
        scratch_types=[
            pltpu.VMEM_SHARED((npad, h), jnp.float32),
            pltpu.VMEM((ch, K), jnp.int32),
            pltpu.VMEM((ch, K), jnp.int32),
            pltpu.VMEM((K, h), jnp.float32),
            pltpu.VMEM((sl, h), jnp.float32),
            pltpu.SemaphoreType.DMA,
        ],
    )
    def seg_kernel(y_hbm, src_hbm, dst_hbm, zeros_hbm, out_hbm,
                   acc_sh, src_v, dst_v, rows_v, zst_v, gsem):
        c = lax.axis_index("c")
        s = lax.axis_index("s")
        wid = s * NC + c
        pltpu.sync_copy(zeros_hbm.at[pl.ds(s * sl, sl)], zst_v)
        pltpu.sync_copy(zst_v, acc_sh.at[pl.ds(s * sl, sl)])
        pltpu.sync_copy(src_hbm.at[wid], src_v)
        pltpu.sync_copy(dst_hbm.at[wid], dst_v)
        plsc.subcore_barrier()

        @pl.loop(0, ch)
        def _(j):
            pltpu.async_copy(y_hbm.at[src_v.at[j]], rows_v, gsem).wait()
            pltpu.sync_copy(rows_v, acc_sh.at[dst_v.at[j]], add=True)

        plsc.subcore_barrier()
        pltpu.sync_copy(acc_sh.at[pl.ds(s * sl, sl)], zst_v)
        pltpu.sync_copy(zst_v, out_hbm.at[c, pl.ds(s * sl, sl)])

    return seg_kernel


def _tc1_body(x_ref, wg_ref, wps_ref, bps_ref, xw_ref, zsem_ref):
    xb = x_ref[...]
    xw_ref[...] = jnp.dot(xb, wg_ref[...], preferred_element_type=jnp.float32)
    zsem_ref[...] = (
        jnp.dot(xb, wps_ref[...], preferred_element_type=jnp.float32) + bps_ref[...]
    )


def _tc2_body(degt_ref, xw_ref, y_ref):
    deg = degt_ref[:, 0:1] + degt_ref[:, 1:2] + 1.0
    dis = lax.rsqrt(deg)
    y_ref[...] = xw_ref[...] * dis


def _tc3_body(degt_ref, ap_ref, xw_ref, zsem_ref, wpt_ref, bpt_ref,
              wcls_ref, bcls_ref, bgcn_ref, zt_ref, lg_ref, an_ref):
    acc = ap_ref[0] + ap_ref[1]
    deg = degt_ref[:, 0:1] + degt_ref[:, 1:2] + 1.0
    dis = lax.rsqrt(deg)
    h = acc * dis + xw_ref[...] * (dis * dis) + bgcn_ref[...]
    h = jnp.maximum(h, 0.0)
    zt = jnp.dot(h, wpt_ref[...], preferred_element_type=jnp.float32) + bpt_ref[...]
    lg = jnp.dot(zt, wcls_ref[...], preferred_element_type=jnp.float32) + bcls_ref[...]
    diff = zt - zsem_ref[...]
    an_ref[...] = jnp.sqrt(jnp.sum(diff * diff, axis=1, keepdims=True))
    zt_ref[...] = zt
    lg_ref[...] = lg


def kernel(x, edge_index, W_gcn, b_gcn, W_pt, b_pt, W_ps, b_ps, W_cls, b_cls):
    n, d = x.shape
    h = W_gcn.shape[1]
    a = W_pt.shape[1]
    ncls = W_cls.shape[1]
    e = edge_index.shape[1]

    npad = ((n + 1 + 127) // 128) * 128
    ch = -(-e // (NW * K))
    ep = NW * ch * K

    src = edge_index[0]
    dst = edge_index[1]
    padi = jnp.full((ep - e,), n, jnp.int32)
    srcp = jnp.concatenate([src, padi]).reshape(NW, ch, K)
    dstp = jnp.concatenate([dst, padi]).reshape(NW, ch, K)
    xp = jnp.concatenate([x, jnp.zeros((npad - n, d), x.dtype)], axis=0)
    zeros_n = jnp.zeros((npad,), jnp.float32)
    zeros_nh = jnp.zeros((npad, h), jnp.float32)

    bn = npad // 8

    xw, z_sem = pl.pallas_call(
        _tc1_body,
        grid=(8,),
        in_specs=[
            pl.BlockSpec((bn, d), lambda i: (i, 0)),
            pl.BlockSpec((d, h), lambda i: (0, 0)),
            pl.BlockSpec((d, a), lambda i: (0, 0)),
            pl.BlockSpec((1, a), lambda i: (0, 0)),
        ],
        out_specs=[
            pl.BlockSpec((bn, h), lambda i: (i, 0)),
            pl.BlockSpec((bn, a), lambda i: (i, 0)),
        ],
        out_shape=[
            jax.ShapeDtypeStruct((npad, h), jnp.float32),
            jax.ShapeDtypeStruct((npad, a), jnp.float32),
        ],
    )(xp, W_gcn, W_ps, b_ps.reshape(1, a))

    deg_parts = _deg_kernel(npad, ch)(dstp, zeros_n)
    deg_t = deg_parts.reshape(NC, npad).T

    y = pl.pallas_call(
        _tc2_body,
        grid=(8,),
        in_specs=[
            pl.BlockSpec((bn, NC), lambda i: (i, 0)),
            pl.BlockSpec((bn, h), lambda i: (i, 0)),
        ],
        out_specs=pl.BlockSpec((bn, h), lambda i: (i, 0)),
        out_shape=jax.ShapeDtypeStruct((npad, h), jnp.float32),
    )(deg_t, xw)

    acc_parts = _segsum_kernel(npad, ch, h)(y, srcp, dstp, zeros_nh)

    z_topo, logits, anomaly = pl.pallas_call(
        _tc3_body,
        grid=(8,),
        in_specs=[
            pl.BlockSpec((bn, NC), lambda i: (i, 0)),
            pl.BlockSpec((NC, bn, h), lambda i: (0, i, 0)),
            pl.BlockSpec((bn, h), lambda i: (i, 0)),
            pl.BlockSpec((bn, a), lambda i: (i, 0)),
            pl.BlockSpec((h, a), lambda i: (0, 0)),
            pl.BlockSpec((1, a), lambda i: (0, 0)),
            pl.BlockSpec((a, ncls), lambda i: (0, 0)),
            pl.BlockSpec((1, ncls), lambda i: (0, 0)),
            pl.BlockSpec((1, h), lambda i: (0, 0)),
        ],
        out_specs=[
            pl.BlockSpec((bn, a), lambda i: (i, 0)),
            pl.BlockSpec((bn, ncls), lambda i: (i, 0)),
            pl.BlockSpec((bn, 1), lambda i: (i, 0)),
        ],
        out_shape=[
            jax.ShapeDtypeStruct((npad, a), jnp.float32),
            jax.ShapeDtypeStruct((npad, ncls), jnp.float32),
            jax.ShapeDtypeStruct((npad, 1), jnp.float32),
        ],
    )(deg_t, acc_parts, xw, z_sem, W_pt, b_pt.reshape(1, a),
      W_cls, b_cls.reshape(1, ncls), b_gcn.reshape(1, h))

    return (logits[:n], anomaly[:n, 0], z_topo[:n], z_sem[:n])
